# Initial kernel scaffold; baseline (speedup 1.0000x reference)
import jax
import jax.numpy as jnp
from jax.experimental import pallas as pl


def kernel(x, node_in, node_out, relation, edge_weight, W_lin, b_lin, W_loop, b_loop):
    raise NotImplementedError("write your pallas kernel here")


# speedup vs baseline: 5.0493x; 5.0493x over previous
import functools

import jax
import jax.numpy as jnp
from jax import lax
from jax.experimental import pallas as pl
from jax.experimental.pallas import tpu as pltpu
from jax.experimental.pallas import tpu_sc as plsc

N = 10000
E = 320000
R = 4
D = 128
OUT = 128
EPS = 1e-10

NC = 2
NS = 16
NW = NC * NS
L = 16

EPW = E // NW
CA = 80
NCHUNK = EPW // CA
NR = N * R
NR_PAD = 40960
DEG_PER_TILE = NR_PAD // NS
N_PAD = 10240
ROWS_PER_TILE = N_PAD // NS
ZR = 128

BN = 1000

_GDN = lax.GatherDimensionNumbers(
    offset_dims=(), collapsed_slice_dims=(0,), start_index_map=(0,))


def _bcast_lane(vec, i):
    idx = jnp.full((L, 1), i, jnp.int32)
    return lax.gather(vec, idx, _GDN, (1,),
                      mode=lax.GatherScatterMode.PROMISE_IN_BOUNDS)


def _y_body(x_ref, w_ref, y_ref):
    y_ref[0] = lax.dot_general(
        x_ref[...], w_ref[0], (((1,), (1,)), ((), ())),
        preferred_element_type=jnp.float32,
        precision=lax.Precision.HIGHEST)


def _final_body(acc_ref, x_ref, wl_ref, bl_ref, bp_ref, o_ref):
    z = acc_ref[0] + acc_ref[1]
    z = z + lax.dot_general(
        x_ref[...], wl_ref[...], (((1,), (1,)), ((), ())),
        preferred_element_type=jnp.float32,
        precision=lax.Precision.HIGHEST)
    z = z + bl_ref[...] + bp_ref[...]
    o_ref[...] = jnp.maximum(z, 0.0)


def _deg_body(no_hbm, rel_hbm, ew_hbm, deg_hbm,
              no_buf, rel_buf, ew_buf, idx_buf, zero_buf, deg_sh):
    cid = lax.axis_index("c")
    sid = lax.axis_index("s")
    wid = cid * NS + sid

    def zstore(i, _):
        zero_buf[pl.ds(i * L, L)] = jnp.zeros((L,), jnp.float32)
        return 0
    lax.fori_loop(0, DEG_PER_TILE // L, zstore, 0)
    pltpu.sync_copy(zero_buf, deg_sh.at[pl.ds(sid * DEG_PER_TILE, DEG_PER_TILE)])
    plsc.subcore_barrier()

    base = wid * EPW

    def chunk(ci, _):
        off = base + ci * CA
        pltpu.sync_copy(no_hbm.at[pl.ds(off, CA)], no_buf)
        pltpu.sync_copy(rel_hbm.at[pl.ds(off, CA)], rel_buf)
        pltpu.sync_copy(ew_hbm.at[pl.ds(off, CA)], ew_buf)

        def v16(i, _):
            sl = pl.ds(i * L, L)
            idx_buf[sl] = no_buf[sl] * R + rel_buf[sl]
            return 0
        lax.fori_loop(0, CA // L, v16, 0)
        pltpu.sync_copy(ew_buf, deg_sh.at[idx_buf], add=True)
        return 0
    lax.fori_loop(0, NCHUNK, chunk, 0)

    plsc.subcore_barrier()
    pltpu.sync_copy(deg_sh.at[pl.ds(sid * DEG_PER_TILE, DEG_PER_TILE)],
                    deg_hbm.at[pl.ds(cid * NR_PAD + sid * DEG_PER_TILE,
                                     DEG_PER_TILE)])


def _agg_body(in_hbm, no_hbm, rel_hbm, ew_hbm, y_hbm, deg0_hbm, deg1_hbm,
              acc_hbm,
              in_buf, no_buf, rel_buf, ew_buf, idx_buf, j_buf,
              d0_buf, d1_buf, nw_buf, rows, zero_buf, acc_sh):
    cid = lax.axis_index("c")
    sid = lax.axis_index("s")
    wid = cid * NS + sid

    def zstore(i, _):
        for v in range(OUT // L):
            zero_buf[i, pl.ds(v * L, L)] = jnp.zeros((L,), jnp.float32)
        return 0
    lax.fori_loop(0, ZR, zstore, 0)
    for k in range(ROWS_PER_TILE // ZR):
        pltpu.sync_copy(zero_buf, acc_sh.at[pl.ds(sid * ROWS_PER_TILE + k * ZR, ZR)])
    plsc.subcore_barrier()

    base = wid * EPW

    def chunk(ci, _):
        off = base + ci * CA
        pltpu.sync_copy(in_hbm.at[pl.ds(off, CA)], in_buf)
        pltpu.sync_copy(no_hbm.at[pl.ds(off, CA)], no_buf)
        pltpu.sync_copy(rel_hbm.at[pl.ds(off, CA)], rel_buf)
        pltpu.sync_copy(ew_hbm.at[pl.ds(off, CA)], ew_buf)

        def vidx(i, _):
            sl = pl.ds(i * L, L)
            rl = rel_buf[sl]
            idx_buf[sl] = no_buf[sl] * R + rl
            j_buf[sl] = rl * N + in_buf[sl]
            return 0
        lax.fori_loop(0, CA // L, vidx, 0)

        pltpu.sync_copy(deg0_hbm.at[idx_buf], d0_buf)
        pltpu.sync_copy(deg1_hbm.at[idx_buf], d1_buf)

        def vnw(i, _):
            sl = pl.ds(i * L, L)
            nw_buf[sl] = ew_buf[sl] / (d0_buf[sl] + d1_buf[sl] + EPS)
            return 0
        lax.fori_loop(0, CA // L, vnw, 0)

        pltpu.sync_copy(y_hbm.at[j_buf], rows)

        def vscale(g, _):
            nwv = nw_buf[pl.ds(g * L, L)]
            for i in range(L):
                k = g * L + i
                s = _bcast_lane(nwv, i)
                for v in range(OUT // L):
                    sl = pl.ds(v * L, L)
                    rows[k, sl] = rows[k, sl] * s
            return 0
        lax.fori_loop(0, CA // L, vscale, 0)

        pltpu.sync_copy(rows, acc_sh.at[no_buf], add=True)
        return 0
    lax.fori_loop(0, NCHUNK, chunk, 0)

    plsc.subcore_barrier()
    for k in range(ROWS_PER_TILE // ZR):
        r0 = sid * ROWS_PER_TILE + k * ZR
        pltpu.sync_copy(acc_sh.at[pl.ds(r0, ZR)], acc_hbm.at[cid, pl.ds(r0, ZR)])


_sc_mesh = plsc.VectorSubcoreMesh(core_axis_name="c", subcore_axis_name="s",
                                  num_cores=NC, num_subcores=NS)

_deg_kernel = functools.partial(
    pl.kernel,
    out_type=jax.ShapeDtypeStruct((NC * NR_PAD,), jnp.float32),
    mesh=_sc_mesh,
    scratch_types=[
        pltpu.VMEM((CA,), jnp.int32),
        pltpu.VMEM((CA,), jnp.int32),
        pltpu.VMEM((CA,), jnp.float32),
        pltpu.VMEM((CA,), jnp.int32),
        pltpu.VMEM((DEG_PER_TILE,), jnp.float32),
        pltpu.VMEM_SHARED((NR_PAD,), jnp.float32),
    ],
)(_deg_body)

_agg_kernel = functools.partial(
    pl.kernel,
    out_type=jax.ShapeDtypeStruct((NC, N_PAD, OUT), jnp.float32),
    mesh=_sc_mesh,
    scratch_types=[
        pltpu.VMEM((CA,), jnp.int32),
        pltpu.VMEM((CA,), jnp.int32),
        pltpu.VMEM((CA,), jnp.int32),
        pltpu.VMEM((CA,), jnp.float32),
        pltpu.VMEM((CA,), jnp.int32),
        pltpu.VMEM((CA,), jnp.int32),
        pltpu.VMEM((CA,), jnp.float32),
        pltpu.VMEM((CA,), jnp.float32),
        pltpu.VMEM((CA,), jnp.float32),
        pltpu.VMEM((CA, OUT), jnp.float32),
        pltpu.VMEM((ZR, OUT), jnp.float32),
        pltpu.VMEM_SHARED((N_PAD, OUT), jnp.float32),
    ],
)(_agg_body)


def kernel(x, node_in, node_out, relation, edge_weight, W_lin, b_lin, W_loop, b_loop):
    node_in = node_in.astype(jnp.int32)
    node_out = node_out.astype(jnp.int32)
    relation = relation.astype(jnp.int32)

    w3 = W_lin.reshape(OUT, R, D).swapaxes(0, 1)
    y = pl.pallas_call(
        _y_body,
        grid=(R, N // BN),
        in_specs=[
            pl.BlockSpec((BN, D), lambda r, n: (n, 0)),
            pl.BlockSpec((1, OUT, D), lambda r, n: (r, 0, 0)),
        ],
        out_specs=pl.BlockSpec((1, BN, OUT), lambda r, n: (r, n, 0)),
        out_shape=jax.ShapeDtypeStruct((R, N, OUT), jnp.float32),
    )(x, w3)
    y_flat = y.reshape(R * N, OUT)

    degp = _deg_kernel(node_out, relation, edge_weight)

    accp = _agg_kernel(node_in, node_out, relation, edge_weight,
                       y_flat, degp[:NR_PAD], degp[NR_PAD:])
    accp = accp[:, :N, :]

    out = pl.pallas_call(
        _final_body,
        grid=(N // BN,),
        in_specs=[
            pl.BlockSpec((2, BN, OUT), lambda n: (0, n, 0)),
            pl.BlockSpec((BN, D), lambda n: (n, 0)),
            pl.BlockSpec((OUT, D), lambda n: (0, 0)),
            pl.BlockSpec((1, OUT), lambda n: (0, 0)),
            pl.BlockSpec((1, OUT), lambda n: (0, 0)),
        ],
        out_specs=pl.BlockSpec((BN, OUT), lambda n: (n, 0)),
        out_shape=jax.ShapeDtypeStruct((N, OUT), jnp.float32),
    )(accp, x, W_loop, b_lin.reshape(1, OUT), b_loop.reshape(1, OUT))
    return out
